# Initial kernel scaffold; baseline (speedup 1.0000x reference)
import jax
import jax.numpy as jnp
from jax.experimental import pallas as pl


def kernel(z_e, codebook):
    raise NotImplementedError("write your pallas kernel here")


# speedup vs baseline: 1.2231x; 1.2231x over previous
import jax
import jax.numpy as jnp
from jax.experimental import pallas as pl


def _vq_body(z_ref, cbT_ref, cb_ref, idx_ref, zq_ref):
    zb = z_ref[...]
    cbT = cbT_ref[...]
    cb = cb_ref[...]
    BT = zb.shape[0]
    K = cb.shape[0]
    cross = jax.lax.dot_general(
        zb, cbT, (((1,), (0,)), ((), ())),
        preferred_element_type=jnp.float32)
    z_sq = jnp.sum(zb * zb, axis=-1, keepdims=True)
    e_sq = jnp.sum(cb * cb, axis=-1)
    dists = z_sq - 2.0 * cross + e_sq[None, :]
    m = jnp.min(dists, axis=-1, keepdims=True)
    kiota = jax.lax.broadcasted_iota(jnp.int32, (BT, K), 1)
    idx = jnp.min(jnp.where(dists == m, kiota, K), axis=-1)
    idx_ref[...] = idx
    onehot = (kiota == idx[:, None]).astype(jnp.float32)
    zq_ref[...] = jax.lax.dot_general(
        onehot, cb, (((1,), (0,)), ((), ())),
        preferred_element_type=jnp.float32)


def kernel(z_e, codebook):
    B, T, D = z_e.shape
    K = codebook.shape[0]
    N = B * T
    z = z_e.reshape(N, D)
    cbT = codebook.T
    BT = 512
    grid = (N // BT,)

    idx_flat, zq_flat = pl.pallas_call(
        _vq_body,
        grid=grid,
        in_specs=[
            pl.BlockSpec((BT, D), lambda i: (i, 0)),
            pl.BlockSpec((D, K), lambda i: (0, 0)),
            pl.BlockSpec((K, D), lambda i: (0, 0)),
        ],
        out_specs=[
            pl.BlockSpec((BT,), lambda i: (i,)),
            pl.BlockSpec((BT, D), lambda i: (i, 0)),
        ],
        out_shape=[
            jax.ShapeDtypeStruct((N,), jnp.int32),
            jax.ShapeDtypeStruct((N, D), jnp.float32),
        ],
    )(z, cbT, codebook)

    return idx_flat.reshape(B, T), zq_flat.reshape(B, T, D)
